# Initial kernel scaffold; baseline (speedup 1.0000x reference)
import functools

import jax
import jax.numpy as jnp
from jax import lax
from jax.experimental import pallas as pl
from jax.experimental.pallas import tpu as pltpu
from jax.experimental.pallas import tpu_sc as plsc

N = 10000
E = 320000
D = 128

NUM_CORES = 2
NUM_SUBCORES = 16
NUM_WORKERS = NUM_CORES * NUM_SUBCORES
EDGES_PER_WORKER = E // NUM_WORKERS
CHUNK = 80
CHUNKS_PER_WORKER = EDGES_PER_WORKER // CHUNK
ROWS_PER_TILE = N // NUM_SUBCORES
ZROWS = 25


def _mm_body(x_ref, w_ref, b_ref, o_ref):
    h = lax.dot_general(x_ref[...], w_ref[...],
                        (((1,), (1,)), ((), ())),
                        preferred_element_type=jnp.float32)
    o_ref[...] = h + b_ref[...]


def _matmul(x2d, W, b2d):
    blk = 1250
    grid = N // blk
    return pl.pallas_call(
        _mm_body,
        grid=(grid,),
        in_specs=[
            pl.BlockSpec((blk, D), lambda i: (i, 0)),
            pl.BlockSpec((D, D), lambda i: (0, 0)),
            pl.BlockSpec((1, D), lambda i: (0, 0)),
        ],
        out_specs=pl.BlockSpec((blk, D), lambda i: (i, 0)),
        out_shape=jax.ShapeDtypeStruct((N, D), jnp.float32),
    )(x2d, W, b2d)


def _edge_body(h_hbm, src_hbm, dst_hbm, w_hbm, out_hbm,
               src_v, dst_v, w_v, rows_v, zbuf, acc_sh, sem):
    cid = lax.axis_index("c")
    sid = lax.axis_index("s")
    wid = sid * NUM_CORES + cid

    for i in range(ZROWS):
        for j in range(D // 16):
            zbuf[i, pl.ds(j * 16, 16)] = jnp.zeros((16,), jnp.float32)

    def _zero_step(t, carry):
        pltpu.sync_copy(zbuf, acc_sh.at[pl.ds(sid * ROWS_PER_TILE + t * ZROWS, ZROWS)])
        return carry
    lax.fori_loop(0, ROWS_PER_TILE // ZROWS, _zero_step, 0)
    plsc.subcore_barrier()

    row0 = wid * CHUNKS_PER_WORKER
    pltpu.sync_copy(src_hbm.at[pl.ds(row0, CHUNKS_PER_WORKER)], src_v)
    pltpu.sync_copy(dst_hbm.at[pl.ds(row0, CHUNKS_PER_WORKER)], dst_v)
    pltpu.sync_copy(w_hbm.at[pl.ds(row0, CHUNKS_PER_WORKER)], w_v)

    def _chunk_step(g, carry):
        pltpu.async_copy(h_hbm.at[src_v.at[g]], rows_v, sem).wait()

        def _edge_step(e, c2):
            w_b = plsc.load_gather(
                w_v, [jnp.full((16,), g, jnp.int32), jnp.full((16,), e, jnp.int32)])
            for j in range(D // 16):
                sl = pl.ds(j * 16, 16)
                rows_v[e, sl] = rows_v[e, sl] * w_b
            return c2
        lax.fori_loop(0, CHUNK, _edge_step, 0)

        pltpu.sync_copy(rows_v, acc_sh.at[dst_v.at[g]], add=True)
        return carry
    lax.fori_loop(0, CHUNKS_PER_WORKER, _chunk_step, 0)
    plsc.subcore_barrier()

    pltpu.sync_copy(acc_sh.at[pl.ds(sid * ROWS_PER_TILE, ROWS_PER_TILE)],
                    out_hbm.at[cid, pl.ds(sid * ROWS_PER_TILE, ROWS_PER_TILE)])


def _edge_pass(h, src2d, dst2d, w2d):
    mesh = plsc.VectorSubcoreMesh(core_axis_name="c", subcore_axis_name="s")
    fn = functools.partial(
        pl.kernel, mesh=mesh,
        out_type=jax.ShapeDtypeStruct((NUM_CORES, N, D), jnp.float32),
        scratch_types=[
            pltpu.VMEM((CHUNKS_PER_WORKER, CHUNK), jnp.int32),
            pltpu.VMEM((CHUNKS_PER_WORKER, CHUNK), jnp.int32),
            pltpu.VMEM((CHUNKS_PER_WORKER, CHUNK), jnp.float32),
            pltpu.VMEM((CHUNK, D), jnp.float32),
            pltpu.VMEM((ZROWS, D), jnp.float32),
            pltpu.VMEM_SHARED((N, D), jnp.float32),
            pltpu.SemaphoreType.DMA,
        ],
    )(_edge_body)
    return fn(h, src2d, dst2d, w2d)


def _comb_body(p_ref, o_ref):
    o_ref[...] = jnp.tanh(p_ref[0] + p_ref[1])


def _combine(partials):
    blk = 1250
    grid = N // blk
    return pl.pallas_call(
        _comb_body,
        grid=(grid,),
        in_specs=[pl.BlockSpec((NUM_CORES, blk, D), lambda i: (0, i, 0))],
        out_specs=pl.BlockSpec((blk, D), lambda i: (i, 0)),
        out_shape=jax.ShapeDtypeStruct((N, D), jnp.float32),
    )(partials)


def kernel(x, edge_index, edge_weight, W, b):
    x2d = x[0]
    b2d = b.reshape(1, D)
    h = _matmul(x2d, W, b2d)
    src2d = edge_index[1].reshape(E // CHUNK, CHUNK)
    dst2d = edge_index[0].reshape(E // CHUNK, CHUNK)
    w2d = edge_weight.reshape(E // CHUNK, CHUNK)
    partials = _edge_pass(h, src2d, dst2d, w2d)
    out = _combine(partials)
    return out[None, :, :]


# speedup vs baseline: 6.1136x; 6.1136x over previous
import functools

import jax
import jax.numpy as jnp
from jax import lax
from jax.experimental import pallas as pl
from jax.experimental.pallas import tpu as pltpu
from jax.experimental.pallas import tpu_sc as plsc

N = 10000
E = 320000
D = 128

NUM_CORES = 2
NUM_SUBCORES = 16
NUM_WORKERS = NUM_CORES * NUM_SUBCORES
EDGES_PER_WORKER = E // NUM_WORKERS
CHUNK = 80
CHUNKS_PER_WORKER = EDGES_PER_WORKER // CHUNK
NPAD = 10112
ROWS_PER_TILE = NPAD // NUM_SUBCORES
ZROWS = 8


def _mm_body(x_ref, w_ref, b_ref, o_ref):
    h = lax.dot_general(x_ref[...], w_ref[...],
                        (((1,), (1,)), ((), ())),
                        preferred_element_type=jnp.float32)
    o_ref[...] = h + b_ref[...]


def _matmul(x2d, W, b2d):
    blk = 1000
    grid = N // blk
    return pl.pallas_call(
        _mm_body,
        grid=(grid,),
        in_specs=[
            pl.BlockSpec((blk, D), lambda i: (i, 0)),
            pl.BlockSpec((D, D), lambda i: (0, 0)),
            pl.BlockSpec((1, D), lambda i: (0, 0)),
        ],
        out_specs=pl.BlockSpec((blk, D), lambda i: (i, 0)),
        out_shape=jax.ShapeDtypeStruct((N, D), jnp.float32),
    )(x2d, W, b2d)


def _edge_body(h_hbm, src_hbm, dst_hbm, w_hbm, out_hbm,
               src_v, dst_v, w_v, dst_c, rows_v, zbuf, acc_sh, sem):
    cid = lax.axis_index("c")
    sid = lax.axis_index("s")
    wid = sid * NUM_CORES + cid

    for i in range(ZROWS):
        for j in range(D // 16):
            zbuf[i, pl.ds(j * 16, 16)] = jnp.zeros((16,), jnp.float32)

    def _zero_step(t, carry):
        pltpu.sync_copy(zbuf, acc_sh.at[pl.ds(sid * ROWS_PER_TILE + t * ZROWS, ZROWS)])
        return carry
    lax.fori_loop(0, ROWS_PER_TILE // ZROWS, _zero_step, 0)
    plsc.subcore_barrier()

    base = wid * EDGES_PER_WORKER
    pltpu.sync_copy(src_hbm.at[pl.ds(base, EDGES_PER_WORKER)], src_v)
    pltpu.sync_copy(dst_hbm.at[pl.ds(base, EDGES_PER_WORKER)], dst_v)
    pltpu.sync_copy(w_hbm.at[pl.ds(base, EDGES_PER_WORKER)], w_v)

    def _chunk_step(g, carry):
        pltpu.async_copy(
            h_hbm.at[src_v.at[pl.ds(g * CHUNK, CHUNK)]], rows_v, sem).wait()

        for q in range(CHUNK // 16):
            dst_c[pl.ds(q * 16, 16)] = dst_v[pl.ds(g * CHUNK + q * 16, 16)]

        for q in range(CHUNK // 16):
            w16 = w_v[pl.ds(g * CHUNK + q * 16, 16)]

            def _edge_step(e16, c2, _w16=w16, _q=q):
                w_b = lax.gather(
                    _w16, jnp.full((16, 1), e16, jnp.int32),
                    lax.GatherDimensionNumbers(
                        offset_dims=(), collapsed_slice_dims=(0,),
                        start_index_map=(0,)),
                    (1,), mode=lax.GatherScatterMode.PROMISE_IN_BOUNDS)
                e = _q * 16 + e16
                for j in range(D // 16):
                    sl = pl.ds(j * 16, 16)
                    rows_v[e, sl] = rows_v[e, sl] * w_b
                return c2
            lax.fori_loop(0, 16, _edge_step, 0)

        pltpu.sync_copy(rows_v, acc_sh.at[dst_c], add=True)
        return carry
    lax.fori_loop(0, CHUNKS_PER_WORKER, _chunk_step, 0)
    plsc.subcore_barrier()

    pltpu.sync_copy(acc_sh.at[pl.ds(sid * ROWS_PER_TILE, ROWS_PER_TILE)],
                    out_hbm.at[cid, pl.ds(sid * ROWS_PER_TILE, ROWS_PER_TILE)])


def _edge_pass(h, src, dst, w):
    mesh = plsc.VectorSubcoreMesh(core_axis_name="c", subcore_axis_name="s")
    fn = functools.partial(
        pl.kernel, mesh=mesh,
        out_type=jax.ShapeDtypeStruct((NUM_CORES, NPAD, D), jnp.float32),
        scratch_types=[
            pltpu.VMEM((EDGES_PER_WORKER,), jnp.int32),
            pltpu.VMEM((EDGES_PER_WORKER,), jnp.int32),
            pltpu.VMEM((EDGES_PER_WORKER,), jnp.float32),
            pltpu.VMEM((CHUNK,), jnp.int32),
            pltpu.VMEM((CHUNK, D), jnp.float32),
            pltpu.VMEM((ZROWS, D), jnp.float32),
            pltpu.VMEM_SHARED((NPAD, D), jnp.float32),
            pltpu.SemaphoreType.DMA,
        ],
    )(_edge_body)
    return fn(h, src, dst, w)


def _comb_body(p_ref, o_ref):
    o_ref[...] = jnp.tanh(p_ref[0] + p_ref[1])


def _combine(partials):
    blk = 1000
    grid = N // blk
    return pl.pallas_call(
        _comb_body,
        grid=(grid,),
        in_specs=[pl.BlockSpec((NUM_CORES, blk, D), lambda i: (0, i, 0))],
        out_specs=pl.BlockSpec((blk, D), lambda i: (i, 0)),
        out_shape=jax.ShapeDtypeStruct((N, D), jnp.float32),
    )(partials)


def kernel(x, edge_index, edge_weight, W, b):
    x2d = x[0]
    b2d = b.reshape(1, D)
    h = _matmul(x2d, W, b2d)
    partials = _edge_pass(h, edge_index[1], edge_index[0], edge_weight)
    out = _combine(partials)
    return out[None, :, :]


# speedup vs baseline: 8.5143x; 1.3927x over previous
import functools

import jax
import jax.numpy as jnp
from jax import lax
from jax.experimental import pallas as pl
from jax.experimental.pallas import tpu as pltpu
from jax.experimental.pallas import tpu_sc as plsc

N = 10000
E = 320000
D = 128

NUM_CORES = 2
NUM_SUBCORES = 16
NUM_WORKERS = NUM_CORES * NUM_SUBCORES
EPW = 10240
CHUNK = 80
NCHUNKS = EPW // CHUNK
SEG_CHUNKS = 32
SEG_EDGES = SEG_CHUNKS * CHUNK
NSEG = NCHUNKS // SEG_CHUNKS
NPAD = 10112
ROWS_PER_TILE = NPAD // NUM_SUBCORES
ZROWS = 8


def _mm_body(x_ref, w_ref, b_ref, o_ref):
    h = lax.dot_general(x_ref[...], w_ref[...],
                        (((1,), (1,)), ((), ())),
                        preferred_element_type=jnp.float32)
    o_ref[...] = h + b_ref[...]


def _matmul(x2d, W, b2d):
    blk = 1000
    grid = N // blk
    return pl.pallas_call(
        _mm_body,
        grid=(grid,),
        in_specs=[
            pl.BlockSpec((blk, D), lambda i: (i, 0)),
            pl.BlockSpec((D, D), lambda i: (0, 0)),
            pl.BlockSpec((1, D), lambda i: (0, 0)),
        ],
        out_specs=pl.BlockSpec((blk, D), lambda i: (i, 0)),
        out_shape=jax.ShapeDtypeStruct((N, D), jnp.float32),
    )(x2d, W, b2d)


def _edge_body(h_hbm, src_hbm, dst_hbm, w_hbm, out_hbm,
               src_a, src_b, dst_a, dst_b, w_a, w_b,
               dst_c, rows0, rows1, zbuf, acc_sh,
               sem_g0, sem_g1, sem_idx):
    cid = lax.axis_index("c")
    sid = lax.axis_index("s")
    wid = sid * NUM_CORES + cid

    for i in range(ZROWS):
        for j in range(D // 16):
            zbuf[i, pl.ds(j * 16, 16)] = jnp.zeros((16,), jnp.float32)

    def _zero_step(t, carry):
        pltpu.sync_copy(zbuf, acc_sh.at[pl.ds(sid * ROWS_PER_TILE + t * ZROWS, ZROWS)])
        return carry
    lax.fori_loop(0, ROWS_PER_TILE // ZROWS, _zero_step, 0)
    plsc.subcore_barrier()

    base = wid * EPW

    def _scale_and_scatter(rows, dstbuf, wbuf, goff):
        for q in range(CHUNK // 16):
            dst_c[pl.ds(q * 16, 16)] = dstbuf[pl.ds(goff + q * 16, 16)]
        for q in range(CHUNK // 16):
            w16 = wbuf[pl.ds(goff + q * 16, 16)]

            def _edge_step(e16, c2, _w16=w16, _q=q):
                w_b = lax.gather(
                    _w16, jnp.full((16, 1), e16, jnp.int32),
                    lax.GatherDimensionNumbers(
                        offset_dims=(), collapsed_slice_dims=(0,),
                        start_index_map=(0,)),
                    (1,), mode=lax.GatherScatterMode.PROMISE_IN_BOUNDS)
                e = _q * 16 + e16
                for j in range(D // 16):
                    sl = pl.ds(j * 16, 16)
                    rows[e, sl] = rows[e, sl] * w_b
                return c2
            lax.fori_loop(0, 16, _edge_step, 0)
        pltpu.sync_copy(rows, acc_sh.at[dst_c], add=True)

    def _start_gather(srcbuf, goff, rows, sem):
        pltpu.async_copy(h_hbm.at[srcbuf.at[pl.ds(goff, CHUNK)]], rows, sem)

    def _wait_gather(rows, sem):
        pltpu.make_async_copy(h_hbm.at[dst_c], rows, sem).wait()

    for s in range(NSEG):
        sbuf, dbuf, wbuf = (src_a, dst_a, w_a) if s % 2 == 0 else (src_b, dst_b, w_b)
        nbuf = (src_b, dst_b, w_b) if s % 2 == 0 else (src_a, dst_a, w_a)
        seg0 = base + s * SEG_EDGES
        if s == 0:
            pltpu.sync_copy(src_hbm.at[pl.ds(seg0, SEG_EDGES)], sbuf)
            pltpu.sync_copy(dst_hbm.at[pl.ds(seg0, SEG_EDGES)], dbuf)
            pltpu.sync_copy(w_hbm.at[pl.ds(seg0, SEG_EDGES)], wbuf)
        handles = []
        if s < NSEG - 1:
            nxt0 = base + (s + 1) * SEG_EDGES
            handles = [
                pltpu.async_copy(src_hbm.at[pl.ds(nxt0, SEG_EDGES)], nbuf[0], sem_idx),
                pltpu.async_copy(dst_hbm.at[pl.ds(nxt0, SEG_EDGES)], nbuf[1], sem_idx),
                pltpu.async_copy(w_hbm.at[pl.ds(nxt0, SEG_EDGES)], nbuf[2], sem_idx),
            ]
        _start_gather(sbuf, 0, rows0, sem_g0)

        def _pair_step(t, carry):
            g0 = 2 * t
            _wait_gather(rows0, sem_g0)
            _start_gather(sbuf, (g0 + 1) * CHUNK, rows1, sem_g1)
            _scale_and_scatter(rows0, dbuf, wbuf, g0 * CHUNK)
            _wait_gather(rows1, sem_g1)
            _start_gather(sbuf, (g0 + 2) * CHUNK, rows0, sem_g0)
            _scale_and_scatter(rows1, dbuf, wbuf, (g0 + 1) * CHUNK)
            return carry
        lax.fori_loop(0, SEG_CHUNKS // 2 - 1, _pair_step, 0)

        glast = SEG_CHUNKS - 2
        _wait_gather(rows0, sem_g0)
        _start_gather(sbuf, (glast + 1) * CHUNK, rows1, sem_g1)
        _scale_and_scatter(rows0, dbuf, wbuf, glast * CHUNK)
        _wait_gather(rows1, sem_g1)
        _scale_and_scatter(rows1, dbuf, wbuf, (glast + 1) * CHUNK)

        for hdl in handles:
            hdl.wait()

    plsc.subcore_barrier()

    pltpu.sync_copy(acc_sh.at[pl.ds(sid * ROWS_PER_TILE, ROWS_PER_TILE)],
                    out_hbm.at[cid, pl.ds(sid * ROWS_PER_TILE, ROWS_PER_TILE)])


def _edge_pass(h, src, dst, w):
    mesh = plsc.VectorSubcoreMesh(core_axis_name="c", subcore_axis_name="s")
    fn = functools.partial(
        pl.kernel, mesh=mesh,
        out_type=jax.ShapeDtypeStruct((NUM_CORES, NPAD, D), jnp.float32),
        scratch_types=[
            pltpu.VMEM((SEG_EDGES,), jnp.int32),
            pltpu.VMEM((SEG_EDGES,), jnp.int32),
            pltpu.VMEM((SEG_EDGES,), jnp.int32),
            pltpu.VMEM((SEG_EDGES,), jnp.int32),
            pltpu.VMEM((SEG_EDGES,), jnp.float32),
            pltpu.VMEM((SEG_EDGES,), jnp.float32),
            pltpu.VMEM((CHUNK,), jnp.int32),
            pltpu.VMEM((CHUNK, D), jnp.float32),
            pltpu.VMEM((CHUNK, D), jnp.float32),
            pltpu.VMEM((ZROWS, D), jnp.float32),
            pltpu.VMEM_SHARED((NPAD, D), jnp.float32),
            pltpu.SemaphoreType.DMA,
            pltpu.SemaphoreType.DMA,
            pltpu.SemaphoreType.DMA,
        ],
    )(_edge_body)
    return fn(h, src, dst, w)


def _comb_body(p_ref, o_ref):
    o_ref[...] = jnp.tanh(p_ref[0] + p_ref[1])


def _combine(partials):
    blk = 1000
    grid = N // blk
    return pl.pallas_call(
        _comb_body,
        grid=(grid,),
        in_specs=[pl.BlockSpec((NUM_CORES, blk, D), lambda i: (0, i, 0))],
        out_specs=pl.BlockSpec((blk, D), lambda i: (i, 0)),
        out_shape=jax.ShapeDtypeStruct((N, D), jnp.float32),
    )(partials)


def _pad_edges(src, dst, w):
    per = E // NUM_WORKERS
    npad = EPW - per
    pad_src = jnp.broadcast_to((jnp.arange(npad, dtype=jnp.int32) * 41) % N,
                               (NUM_WORKERS, npad))
    pad_dst = jnp.broadcast_to(N + (jnp.arange(npad, dtype=jnp.int32) % (NPAD - N)),
                               (NUM_WORKERS, npad))
    pad_w = jnp.zeros((NUM_WORKERS, npad), jnp.float32)
    src2 = jnp.concatenate([src.reshape(NUM_WORKERS, per), pad_src], axis=1)
    dst2 = jnp.concatenate([dst.reshape(NUM_WORKERS, per), pad_dst], axis=1)
    w2 = jnp.concatenate([w.reshape(NUM_WORKERS, per), pad_w], axis=1)
    return src2.reshape(-1), dst2.reshape(-1), w2.reshape(-1)


def kernel(x, edge_index, edge_weight, W, b):
    x2d = x[0]
    b2d = b.reshape(1, D)
    h = _matmul(x2d, W, b2d)
    src, dst, w = _pad_edges(edge_index[1], edge_index[0], edge_weight)
    partials = _edge_pass(h, src, dst, w)
    out = _combine(partials)
    return out[None, :, :]


# speedup vs baseline: 9.4998x; 1.1157x over previous
import functools

import jax
import jax.numpy as jnp
from jax import lax
from jax.experimental import pallas as pl
from jax.experimental.pallas import tpu as pltpu
from jax.experimental.pallas import tpu_sc as plsc

N = 10000
E = 320000
D = 128

NUM_CORES = 2
NUM_SUBCORES = 16
NUM_WORKERS = NUM_CORES * NUM_SUBCORES
EPW = 10560
CHUNK = 80
NCHUNKS = EPW // CHUNK
SEG_CHUNKS = 33
SEG_EDGES = SEG_CHUNKS * CHUNK
NSEG = NCHUNKS // SEG_CHUNKS
NPAD = 10112
ROWS_PER_TILE = NPAD // NUM_SUBCORES
ZROWS = 8


def _mm_body(x_ref, w_ref, b_ref, o_ref):
    h = lax.dot_general(x_ref[...], w_ref[...],
                        (((1,), (1,)), ((), ())),
                        preferred_element_type=jnp.float32)
    o_ref[...] = h + b_ref[...]


def _matmul(x2d, W, b2d):
    blk = 1000
    grid = N // blk
    return pl.pallas_call(
        _mm_body,
        grid=(grid,),
        in_specs=[
            pl.BlockSpec((blk, D), lambda i: (i, 0)),
            pl.BlockSpec((D, D), lambda i: (0, 0)),
            pl.BlockSpec((1, D), lambda i: (0, 0)),
        ],
        out_specs=pl.BlockSpec((blk, D), lambda i: (i, 0)),
        out_shape=jax.ShapeDtypeStruct((N, D), jnp.float32),
    )(x2d, W, b2d)


def _edge_body(h_hbm, src_hbm, dst_hbm, w_hbm, out_hbm,
               src_a, src_b, dst_a, dst_b, w_a, w_b,
               dc0, dc1, dc2, r0, r1, r2, zbuf, acc_sh,
               sg0, sg1, sg2, ss0, ss1, ss2, sem_idx):
    cid = lax.axis_index("c")
    sid = lax.axis_index("s")
    wid = sid * NUM_CORES + cid

    rows = (r0, r1, r2)
    dcs = (dc0, dc1, dc2)
    gsems = (sg0, sg1, sg2)
    ssems = (ss0, ss1, ss2)

    for i in range(ZROWS):
        for j in range(D // 16):
            zbuf[i, pl.ds(j * 16, 16)] = jnp.zeros((16,), jnp.float32)

    def _zero_step(t, carry):
        pltpu.sync_copy(zbuf, acc_sh.at[pl.ds(sid * ROWS_PER_TILE + t * ZROWS, ZROWS)])
        return carry
    lax.fori_loop(0, ROWS_PER_TILE // ZROWS, _zero_step, 0)
    plsc.subcore_barrier()

    base = wid * EPW

    def _scale(slot, wbuf, goff):
        for q in range(CHUNK // 16):
            w16 = wbuf[pl.ds(goff + q * 16, 16)]

            def _edge_step(e16, c2, _w16=w16, _q=q):
                w_b = lax.gather(
                    _w16, jnp.full((16, 1), e16, jnp.int32),
                    lax.GatherDimensionNumbers(
                        offset_dims=(), collapsed_slice_dims=(0,),
                        start_index_map=(0,)),
                    (1,), mode=lax.GatherScatterMode.PROMISE_IN_BOUNDS)
                e = _q * 16 + e16
                for j in range(D // 16):
                    sl = pl.ds(j * 16, 16)
                    rows[slot][e, sl] = rows[slot][e, sl] * w_b
                return c2
            lax.fori_loop(0, 16, _edge_step, 0)

    def _fill_dc(slot, dbuf, goff):
        for q in range(CHUNK // 16):
            dcs[slot][pl.ds(q * 16, 16)] = dbuf[pl.ds(goff + q * 16, 16)]

    def _start_gather(sbuf, goff, slot):
        pltpu.async_copy(h_hbm.at[sbuf.at[pl.ds(goff, CHUNK)]], rows[slot], gsems[slot])

    def _wait_gather(slot):
        pltpu.make_async_copy(h_hbm.at[dcs[slot]], rows[slot], gsems[slot]).wait()

    def _start_scatter(slot):
        pltpu.async_copy(rows[slot], acc_sh.at[dcs[slot]], ssems[slot], add=True)

    def _wait_scatter(slot):
        pltpu.make_async_copy(rows[slot], acc_sh.at[dcs[slot]], ssems[slot]).wait()

    def _body(sbuf, dbuf, wbuf, g, slot, issue_next, wait_prev_scat):
        goff = g * CHUNK
        _wait_gather(slot)
        _fill_dc(slot, dbuf, goff)
        _scale(slot, wbuf, goff)
        _start_scatter(slot)
        nslot = (slot + 2) % 3
        if wait_prev_scat:
            _wait_scatter(nslot)
        if issue_next:
            _start_gather(sbuf, (g + 2) * CHUNK, nslot)

    for s in range(NSEG):
        sbuf, dbuf, wbuf = (src_a, dst_a, w_a) if s % 2 == 0 else (src_b, dst_b, w_b)
        nbuf = (src_b, dst_b, w_b) if s % 2 == 0 else (src_a, dst_a, w_a)
        seg0 = base + s * SEG_EDGES
        if s == 0:
            pltpu.sync_copy(src_hbm.at[pl.ds(seg0, SEG_EDGES)], sbuf)
            pltpu.sync_copy(dst_hbm.at[pl.ds(seg0, SEG_EDGES)], dbuf)
            pltpu.sync_copy(w_hbm.at[pl.ds(seg0, SEG_EDGES)], wbuf)
        handles = []
        if s < NSEG - 1:
            nxt0 = base + (s + 1) * SEG_EDGES
            handles = [
                pltpu.async_copy(src_hbm.at[pl.ds(nxt0, SEG_EDGES)], nbuf[0], sem_idx),
                pltpu.async_copy(dst_hbm.at[pl.ds(nxt0, SEG_EDGES)], nbuf[1], sem_idx),
                pltpu.async_copy(w_hbm.at[pl.ds(nxt0, SEG_EDGES)], nbuf[2], sem_idx),
            ]

        _start_gather(sbuf, 0, 0)
        _start_gather(sbuf, CHUNK, 1)
        _body(sbuf, dbuf, wbuf, 0, 0, True, False)
        _body(sbuf, dbuf, wbuf, 1, 1, True, True)
        _body(sbuf, dbuf, wbuf, 2, 2, True, True)

        def _triple(t, carry):
            a = 3 * t + 3
            _body(sbuf, dbuf, wbuf, a, 0, True, True)
            _body(sbuf, dbuf, wbuf, a + 1, 1, True, True)
            _body(sbuf, dbuf, wbuf, a + 2, 2, True, True)
            return carry
        lax.fori_loop(0, (SEG_CHUNKS - 6) // 3, _triple, 0)

        _body(sbuf, dbuf, wbuf, SEG_CHUNKS - 3, 0, True, True)
        _body(sbuf, dbuf, wbuf, SEG_CHUNKS - 2, 1, False, False)
        _body(sbuf, dbuf, wbuf, SEG_CHUNKS - 1, 2, False, False)
        _wait_scatter(0)
        _wait_scatter(1)
        _wait_scatter(2)

        for hdl in handles:
            hdl.wait()

    plsc.subcore_barrier()

    pltpu.sync_copy(acc_sh.at[pl.ds(sid * ROWS_PER_TILE, ROWS_PER_TILE)],
                    out_hbm.at[cid, pl.ds(sid * ROWS_PER_TILE, ROWS_PER_TILE)])


def _edge_pass(h, src, dst, w):
    mesh = plsc.VectorSubcoreMesh(core_axis_name="c", subcore_axis_name="s")
    fn = functools.partial(
        pl.kernel, mesh=mesh,
        out_type=jax.ShapeDtypeStruct((NUM_CORES, NPAD, D), jnp.float32),
        scratch_types=[
            pltpu.VMEM((SEG_EDGES,), jnp.int32),
            pltpu.VMEM((SEG_EDGES,), jnp.int32),
            pltpu.VMEM((SEG_EDGES,), jnp.int32),
            pltpu.VMEM((SEG_EDGES,), jnp.int32),
            pltpu.VMEM((SEG_EDGES,), jnp.float32),
            pltpu.VMEM((SEG_EDGES,), jnp.float32),
            pltpu.VMEM((CHUNK,), jnp.int32),
            pltpu.VMEM((CHUNK,), jnp.int32),
            pltpu.VMEM((CHUNK,), jnp.int32),
            pltpu.VMEM((CHUNK, D), jnp.float32),
            pltpu.VMEM((CHUNK, D), jnp.float32),
            pltpu.VMEM((CHUNK, D), jnp.float32),
            pltpu.VMEM((ZROWS, D), jnp.float32),
            pltpu.VMEM_SHARED((NPAD, D), jnp.float32),
            pltpu.SemaphoreType.DMA,
            pltpu.SemaphoreType.DMA,
            pltpu.SemaphoreType.DMA,
            pltpu.SemaphoreType.DMA,
            pltpu.SemaphoreType.DMA,
            pltpu.SemaphoreType.DMA,
            pltpu.SemaphoreType.DMA,
        ],
    )(_edge_body)
    return fn(h, src, dst, w)


def _comb_body(p_ref, o_ref):
    o_ref[...] = jnp.tanh(p_ref[0] + p_ref[1])


def _combine(partials):
    blk = 1000
    grid = N // blk
    return pl.pallas_call(
        _comb_body,
        grid=(grid,),
        in_specs=[pl.BlockSpec((NUM_CORES, blk, D), lambda i: (0, i, 0))],
        out_specs=pl.BlockSpec((blk, D), lambda i: (i, 0)),
        out_shape=jax.ShapeDtypeStruct((N, D), jnp.float32),
    )(partials)


def _pad_edges(src, dst, w):
    per = E // NUM_WORKERS
    npad = EPW - per
    pad_src = jnp.broadcast_to((jnp.arange(npad, dtype=jnp.int32) * 41) % N,
                               (NUM_WORKERS, npad))
    pad_dst = jnp.broadcast_to(N + (jnp.arange(npad, dtype=jnp.int32) % (NPAD - N)),
                               (NUM_WORKERS, npad))
    pad_w = jnp.zeros((NUM_WORKERS, npad), jnp.float32)
    src2 = jnp.concatenate([src.reshape(NUM_WORKERS, per), pad_src], axis=1)
    dst2 = jnp.concatenate([dst.reshape(NUM_WORKERS, per), pad_dst], axis=1)
    w2 = jnp.concatenate([w.reshape(NUM_WORKERS, per), pad_w], axis=1)
    return src2.reshape(-1), dst2.reshape(-1), w2.reshape(-1)


def kernel(x, edge_index, edge_weight, W, b):
    x2d = x[0]
    b2d = b.reshape(1, D)
    h = _matmul(x2d, W, b2d)
    src, dst, w = _pad_edges(edge_index[1], edge_index[0], edge_weight)
    partials = _edge_pass(h, src, dst, w)
    out = _combine(partials)
    return out[None, :, :]
